# baseline (speedup 1.0000x reference)
import functools

import jax
import jax.numpy as jnp
from jax import lax
from jax.experimental import pallas as pl
from jax.experimental.pallas import tpu as pltpu
from jax.experimental.pallas import tpu_sc as plsc

B = 4096
S = 200
D = 64
NW = 32
BC = B // NW
NBUF = 4
LOOKAHEAD = 2


def _embedder(idx_hbm, tok_hbm, pos_hbm, out_hbm, idx_v, pos_v, buf, gsem, osem):
    wid = lax.axis_index("s") * 2 + lax.axis_index("c")
    b0 = wid * BC
    pltpu.sync_copy(pos_hbm, pos_v)
    pltpu.sync_copy(idx_hbm.at[:, pl.ds(b0, BC)], idx_v)

    def start_gather(i, slot):
        pltpu.async_copy(tok_hbm.at[idx_v.at[i]], buf.at[slot], gsem.at[slot])

    def start_store(i, slot):
        pltpu.async_copy(buf.at[slot], out_hbm.at[i, pl.ds(b0, BC)], osem.at[slot])

    def wait_gather(i, slot):
        pltpu.make_async_copy(tok_hbm.at[idx_v.at[i]], buf.at[slot], gsem.at[slot]).wait()

    def wait_store(i, slot):
        pltpu.make_async_copy(
            buf.at[slot], out_hbm.at[i, pl.ds(b0, BC)], osem.at[slot]
        ).wait()

    for j in range(LOOKAHEAD):
        start_gather(j, j)

    def body(i, carry):
        s = lax.rem(i, NBUF)
        j = i + LOOKAHEAD
        sj = lax.rem(j, NBUF)

        @pl.when(j < S)
        def _prefetch():
            @pl.when(j >= NBUF)
            def _drain():
                wait_store(j - NBUF, sj)

            start_gather(j, sj)

        wait_gather(i, s)

        p0 = pos_v[i, pl.ds(0, 16)]
        p1 = pos_v[i, pl.ds(16, 16)]
        p2 = pos_v[i, pl.ds(32, 16)]
        p3 = pos_v[i, pl.ds(48, 16)]

        def add_row(r):
            plsc.addupdate(buf.at[s, r, pl.ds(0, 16)], p0)
            plsc.addupdate(buf.at[s, r, pl.ds(16, 16)], p1)
            plsc.addupdate(buf.at[s, r, pl.ds(32, 16)], p2)
            plsc.addupdate(buf.at[s, r, pl.ds(48, 16)], p3)

        plsc.parallel_loop(0, BC, 1, unroll=8)(add_row)
        start_store(i, s)
        return carry

    lax.fori_loop(0, S, body, 0)

    for i in range(S - NBUF, S):
        wait_store(i, i % NBUF)


@jax.jit
def _run(idx_t, token_table, pos_table):
    kern = pl.kernel(
        _embedder,
        out_type=jax.ShapeDtypeStruct((S, B, D), jnp.float32),
        mesh=plsc.VectorSubcoreMesh(core_axis_name="c", subcore_axis_name="s"),
        scratch_types=[
            pltpu.VMEM((S, BC), jnp.int32),
            pltpu.VMEM((S, D), jnp.float32),
            pltpu.VMEM((NBUF, BC, D), jnp.float32),
            pltpu.SemaphoreType.DMA((NBUF,)),
            pltpu.SemaphoreType.DMA((NBUF,)),
        ],
        compiler_params=pltpu.CompilerParams(use_tc_tiling_on_sc=False),
    )
    out = kern(idx_t, token_table, pos_table)
    return out.transpose(1, 0, 2) + 0.0


def kernel(input, token_table, pos_table):
    idx_t = jnp.maximum(input.T.astype(jnp.int32), 0)
    return _run(idx_t, token_table, pos_table)
